# baseline (speedup 1.0000x reference)
import jax
import jax.numpy as jnp
from jax import lax
from jax.experimental import pallas as pl
from jax.experimental.pallas import tpu as pltpu

B, I, F = 4096, 16, 512
SCB = 8
SMALL_N = 8
HEAD_B = SCB * SMALL_N
CB = 64
NBUF = 6
NCHUNK = (B - HEAD_B) // CB
NOUTER = NCHUNK // NBUF
TAIL = NCHUNK - NOUTER * NBUF
assert HEAD_B + NCHUNK * CB == B


def _ring_kernel(o_hbm, l_hbm, imp_ref, o_ref, *scr):
    sob = scr[0:SMALL_N]
    slb = scr[SMALL_N:2 * SMALL_N]
    sos = scr[2 * SMALL_N:3 * SMALL_N]
    sls = scr[3 * SMALL_N:4 * SMALL_N]
    n0 = 4 * SMALL_N
    obufs = scr[n0:n0 + NBUF]
    lbufs = scr[n0 + NBUF:n0 + 2 * NBUF]
    osems = scr[n0 + 2 * NBUF:n0 + 3 * NBUF]
    lsems = scr[n0 + 3 * NBUF:n0 + 4 * NBUF]

    imp = imp_ref[...]

    for s in range(SMALL_N):
        boff = s * SCB
        pltpu.async_copy(o_hbm.at[pl.ds(boff, SCB)], sob[s], sos[s])
        pltpu.async_copy(l_hbm.at[pl.ds(boff, SCB)], slb[s], sls[s])
    for s in range(NBUF):
        boff = HEAD_B + s * CB
        pltpu.async_copy(o_hbm.at[pl.ds(boff, CB)], obufs[s], osems[s])
        pltpu.async_copy(l_hbm.at[pl.ds(boff, CB)], lbufs[s], lsems[s])

    acc = jnp.zeros((I, F), jnp.float32)

    for s in range(SMALL_N):
        boff = s * SCB
        pltpu.make_async_copy(o_hbm.at[pl.ds(boff, SCB)], sob[s], sos[s]).wait()
        pltpu.make_async_copy(l_hbm.at[pl.ds(boff, SCB)], slb[s], sls[s]).wait()

        def sb_body(b, a, ob=sob[s], lb=slb[s]):
            d = imp * (jnp.abs(lb[b]) - ob[b])
            return a + d * d

        acc = lax.fori_loop(0, SCB, sb_body, acc, unroll=2)

    def outer_body(c0, acc):
        for s in range(NBUF):
            c = c0 * NBUF + s
            boff = HEAD_B + c * CB
            ob, lb = obufs[s], lbufs[s]
            pltpu.make_async_copy(o_hbm.at[pl.ds(boff, CB)], ob, osems[s]).wait()
            pltpu.make_async_copy(l_hbm.at[pl.ds(boff, CB)], lb, lsems[s]).wait()

            def b_body(b, a, ob=ob, lb=lb):
                d = imp * (jnp.abs(lb[b]) - ob[b])
                return a + d * d

            acc = lax.fori_loop(0, CB, b_body, acc, unroll=2)

            @pl.when(c + NBUF < NCHUNK)
            def _():
                boff2 = boff + NBUF * CB
                pltpu.async_copy(o_hbm.at[pl.ds(boff2, CB)], obufs[s], osems[s])
                pltpu.async_copy(l_hbm.at[pl.ds(boff2, CB)], lbufs[s], lsems[s])
        return acc

    acc = lax.fori_loop(0, NOUTER, outer_body, acc)

    for s in range(TAIL):
        c = NOUTER * NBUF + s
        boff = HEAD_B + c * CB
        ob, lb = obufs[s], lbufs[s]
        pltpu.make_async_copy(o_hbm.at[pl.ds(boff, CB)], ob, osems[s]).wait()
        pltpu.make_async_copy(l_hbm.at[pl.ds(boff, CB)], lb, lsems[s]).wait()

        def tb_body(b, a, ob=ob, lb=lb):
            d = imp * (jnp.abs(lb[b]) - ob[b])
            return a + d * d

        acc = lax.fori_loop(0, CB, tb_body, acc, unroll=2)

    o_ref[0, :] = jnp.sum(acc, axis=1) * (1.0 / (B * F))


def kernel(out, labels, importance):
    scratch = (
        [pltpu.VMEM((SCB, I, F), jnp.float32) for _ in range(2 * SMALL_N)]
        + [pltpu.SemaphoreType.DMA for _ in range(2 * SMALL_N)]
        + [pltpu.VMEM((CB, I, F), jnp.float32) for _ in range(2 * NBUF)]
        + [pltpu.SemaphoreType.DMA for _ in range(2 * NBUF)]
    )
    res = pl.pallas_call(
        _ring_kernel,
        in_specs=[
            pl.BlockSpec(memory_space=pl.ANY),
            pl.BlockSpec(memory_space=pl.ANY),
            pl.BlockSpec((I, F), lambda: (0, 0)),
        ],
        out_specs=pl.BlockSpec((1, I), lambda: (0, 0)),
        out_shape=jax.ShapeDtypeStruct((1, I), jnp.float32),
        scratch_shapes=scratch,
    )(out, labels, importance)
    return res[0]


# speedup vs baseline: 1.0067x; 1.0067x over previous
import jax
import jax.numpy as jnp
from jax import lax
from jax.experimental import pallas as pl
from jax.experimental.pallas import tpu as pltpu

B, I, F = 4096, 16, 512
SCB = 8
SMALL_N = 0
HEAD_B = SCB * SMALL_N
CB = 64
NBUF = 3
NCHUNK = (B - HEAD_B) // CB
NOUTER = NCHUNK // NBUF
TAIL = NCHUNK - NOUTER * NBUF
assert HEAD_B + NCHUNK * CB == B


def _ring_kernel(o_hbm, l_hbm, imp_ref, o_ref, *scr):
    sob = scr[0:SMALL_N]
    slb = scr[SMALL_N:2 * SMALL_N]
    sos = scr[2 * SMALL_N:3 * SMALL_N]
    sls = scr[3 * SMALL_N:4 * SMALL_N]
    n0 = 4 * SMALL_N
    obufs = scr[n0:n0 + NBUF]
    lbufs = scr[n0 + NBUF:n0 + 2 * NBUF]
    osems = scr[n0 + 2 * NBUF:n0 + 3 * NBUF]
    lsems = scr[n0 + 3 * NBUF:n0 + 4 * NBUF]

    imp = imp_ref[...]

    for s in range(SMALL_N):
        boff = s * SCB
        pltpu.async_copy(o_hbm.at[pl.ds(boff, SCB)], sob[s], sos[s])
        pltpu.async_copy(l_hbm.at[pl.ds(boff, SCB)], slb[s], sls[s])
    for s in range(NBUF):
        boff = HEAD_B + s * CB
        pltpu.async_copy(o_hbm.at[pl.ds(boff, CB)], obufs[s], osems[s])
        pltpu.async_copy(l_hbm.at[pl.ds(boff, CB)], lbufs[s], lsems[s])

    acc = jnp.zeros((I, F), jnp.float32)

    for s in range(SMALL_N):
        boff = s * SCB
        pltpu.make_async_copy(o_hbm.at[pl.ds(boff, SCB)], sob[s], sos[s]).wait()
        pltpu.make_async_copy(l_hbm.at[pl.ds(boff, SCB)], slb[s], sls[s]).wait()

        def sb_body(b, a, ob=sob[s], lb=slb[s]):
            d = imp * (jnp.abs(lb[b]) - ob[b])
            return a + d * d

        acc = lax.fori_loop(0, SCB, sb_body, acc, unroll=2)

    def outer_body(c0, acc):
        for s in range(NBUF):
            c = c0 * NBUF + s
            boff = HEAD_B + c * CB
            ob, lb = obufs[s], lbufs[s]
            pltpu.make_async_copy(o_hbm.at[pl.ds(boff, CB)], ob, osems[s]).wait()
            pltpu.make_async_copy(l_hbm.at[pl.ds(boff, CB)], lb, lsems[s]).wait()

            def b_body(b, a, ob=ob, lb=lb):
                d = imp * (jnp.abs(lb[b]) - ob[b])
                return a + d * d

            acc = lax.fori_loop(0, CB, b_body, acc, unroll=2)

            @pl.when(c + NBUF < NCHUNK)
            def _():
                boff2 = boff + NBUF * CB
                pltpu.async_copy(o_hbm.at[pl.ds(boff2, CB)], obufs[s], osems[s])
                pltpu.async_copy(l_hbm.at[pl.ds(boff2, CB)], lbufs[s], lsems[s])
        return acc

    acc = lax.fori_loop(0, NOUTER, outer_body, acc)

    for s in range(TAIL):
        c = NOUTER * NBUF + s
        boff = HEAD_B + c * CB
        ob, lb = obufs[s], lbufs[s]
        pltpu.make_async_copy(o_hbm.at[pl.ds(boff, CB)], ob, osems[s]).wait()
        pltpu.make_async_copy(l_hbm.at[pl.ds(boff, CB)], lb, lsems[s]).wait()

        def tb_body(b, a, ob=ob, lb=lb):
            d = imp * (jnp.abs(lb[b]) - ob[b])
            return a + d * d

        acc = lax.fori_loop(0, CB, tb_body, acc, unroll=2)

    o_ref[0, :] = jnp.sum(acc, axis=1) * (1.0 / (B * F))


def kernel(out, labels, importance):
    scratch = (
        [pltpu.VMEM((SCB, I, F), jnp.float32) for _ in range(2 * SMALL_N)]
        + [pltpu.SemaphoreType.DMA for _ in range(2 * SMALL_N)]
        + [pltpu.VMEM((CB, I, F), jnp.float32) for _ in range(2 * NBUF)]
        + [pltpu.SemaphoreType.DMA for _ in range(2 * NBUF)]
    )
    res = pl.pallas_call(
        _ring_kernel,
        in_specs=[
            pl.BlockSpec(memory_space=pl.ANY),
            pl.BlockSpec(memory_space=pl.ANY),
            pl.BlockSpec((I, F), lambda: (0, 0)),
        ],
        out_specs=pl.BlockSpec((1, I), lambda: (0, 0)),
        out_shape=jax.ShapeDtypeStruct((1, I), jnp.float32),
        scratch_shapes=scratch,
    )(out, labels, importance)
    return res[0]
